# baseline (speedup 1.0000x reference)
import functools

import jax
import jax.numpy as jnp
from jax import lax
from jax.experimental import pallas as pl
from jax.experimental.pallas import tpu as pltpu
from jax.experimental.pallas import tpu_sc as plsc

_K = 8192
_D = 32
_N = 8192
_TB = 256
_NB = _N // _TB
_COMMIT = 0.25


_NBLK = 4
_BLK = _K // _NBLK


def _argmin_body(x_ref, w_ref, idx_ref, loss_ref):
    i = pl.program_id(0)
    xb = x_ref[...]
    w = w_ref[...]
    xb16 = xb.astype(jnp.bfloat16)
    s = jnp.sum(xb * xb, axis=1)
    wsq = jnp.sum(w * w, axis=1)
    mm = lax.dot_general(xb16, w, (((1,), (1,)), ((), ())),
                         preferred_element_type=jnp.float32)
    dist = (s[:, None] + wsq[None, :]) - 2.0 * mm
    accv = acci = vsel = None
    for b in range(_NBLK):
        db = dist[:, b * _BLK:(b + 1) * _BLK]
        mv = jnp.min(db, axis=1)
        ids = lax.broadcasted_iota(jnp.int32, (_TB, _BLK), 1) + (b * _BLK)
        im = jnp.min(jnp.where(db == mv[:, None], ids, jnp.int32(2**31 - 1)),
                     axis=1)
        mvb = mv.astype(jnp.bfloat16).astype(jnp.float32)
        if b == 0:
            accv, acci, vsel = mvb, im, mv
        else:
            repl = mv < accv
            accv = jnp.where(repl, mvb, accv)
            acci = jnp.where(repl, im, acci)
            vsel = jnp.where(repl, mv, vsel)
    idx_ref[0, 0, :] = acci

    @pl.when(i == 0)
    def _():
        loss_ref[...] = jnp.zeros((1, 1), jnp.float32)

    loss_ref[...] += jnp.sum(vsel.reshape(1, _TB), axis=1, keepdims=True)


_argmin_call = pl.pallas_call(
    _argmin_body,
    grid=(_NB,),
    in_specs=[
        pl.BlockSpec((_TB, _D), lambda i: (i, 0)),
        pl.BlockSpec((_K, _D), lambda i: (0, 0)),
    ],
    out_specs=[
        pl.BlockSpec((1, 1, _TB), lambda i: (i, 0, 0)),
        pl.BlockSpec((1, 1), lambda i: (0, 0)),
    ],
    out_shape=[
        jax.ShapeDtypeStruct((_NB, 1, _TB), jnp.int32),
        jax.ShapeDtypeStruct((1, 1), jnp.float32),
    ],
)


_NC = 2
_NS = 16
_NW = _NC * _NS
_BPW = _N // _NW
_CH = 128
_NCH = _BPW // _CH


def _gather_body(w_hbm, idx_hbm, out_hbm, idx_v, rows_v, sem):
    wid = lax.axis_index("s") * _NC + lax.axis_index("c")
    base = wid * _BPW
    for j in range(_NCH):
        pltpu.sync_copy(idx_hbm.at[pl.ds(base + j * _CH, _CH)], idx_v.at[j])
    copies = [
        pltpu.async_copy(w_hbm.at[idx_v.at[j]],
                         rows_v.at[pl.ds(j * _CH, _CH)], sem)
        for j in range(_NCH)
    ]
    for c in copies:
        c.wait()
    pltpu.sync_copy(rows_v, out_hbm.at[pl.ds(base, _BPW)])


@functools.cache
def _make_gather_call():
    return pl.kernel(
        _gather_body,
        mesh=plsc.VectorSubcoreMesh(core_axis_name="c", subcore_axis_name="s"),
        compiler_params=pltpu.CompilerParams(use_tc_tiling_on_sc=False),
        out_type=jax.ShapeDtypeStruct((_N, _D), jnp.float32),
        scratch_types=[
            pltpu.VMEM((_NCH, _CH), jnp.int32),
            pltpu.VMEM((_BPW, _D), jnp.float32),
            pltpu.SemaphoreType.DMA,
        ],
    )


def kernel(x, W):
    xf = x.reshape(_N, _D)
    idx3, loss_sum = _argmin_call(xf, W)
    idx = idx3.reshape(_N)
    q = _make_gather_call()(W, idx)
    quantized = x + (q.reshape(x.shape) - x)
    mean_sq = loss_sum[0, 0] / (_N * _D)
    loss = mean_sq + _COMMIT * mean_sq
    return quantized, loss, idx.reshape(_N, 1)


# speedup vs baseline: 1.1754x; 1.1754x over previous
import functools

import jax
import jax.numpy as jnp
from jax import lax
from jax.experimental import pallas as pl
from jax.experimental.pallas import tpu as pltpu
from jax.experimental.pallas import tpu_sc as plsc

_K = 8192
_D = 32
_N = 8192
_TB = 512
_NB = _N // _TB
_COMMIT = 0.25


_NBLK = 4
_BLK = _K // _NBLK


def _argmin_body(x_ref, w_ref, idx_ref, loss_ref):
    i = pl.program_id(0)
    xb = x_ref[...]
    w = w_ref[...]
    xb16 = xb.astype(jnp.bfloat16)
    s = jnp.sum(xb * xb, axis=1)
    wsq = jnp.sum(w * w, axis=1)
    mm = lax.dot_general(xb16, w, (((1,), (1,)), ((), ())),
                         preferred_element_type=jnp.float32)
    dist = (s[:, None] + wsq[None, :]) - 2.0 * mm
    accv = acci = vsel = None
    for b in range(_NBLK):
        db = dist[:, b * _BLK:(b + 1) * _BLK]
        mv = jnp.min(db, axis=1)
        ids = lax.broadcasted_iota(jnp.int32, (_TB, _BLK), 1) + (b * _BLK)
        im = jnp.min(jnp.where(db == mv[:, None], ids, jnp.int32(2**31 - 1)),
                     axis=1)
        mvb = mv.astype(jnp.bfloat16).astype(jnp.float32)
        if b == 0:
            accv, acci, vsel = mvb, im, mv
        else:
            repl = mv < accv
            accv = jnp.where(repl, mvb, accv)
            acci = jnp.where(repl, im, acci)
            vsel = jnp.where(repl, mv, vsel)
    idx_ref[0, 0, :] = acci

    @pl.when(i == 0)
    def _():
        loss_ref[...] = jnp.zeros((1, 1), jnp.float32)

    loss_ref[...] += jnp.sum(vsel.reshape(1, _TB), axis=1, keepdims=True)


_argmin_call = pl.pallas_call(
    _argmin_body,
    grid=(_NB,),
    in_specs=[
        pl.BlockSpec((_TB, _D), lambda i: (i, 0)),
        pl.BlockSpec((_K, _D), lambda i: (0, 0)),
    ],
    out_specs=[
        pl.BlockSpec((1, 1, _TB), lambda i: (i, 0, 0)),
        pl.BlockSpec((1, 1), lambda i: (0, 0)),
    ],
    out_shape=[
        jax.ShapeDtypeStruct((_NB, 1, _TB), jnp.int32),
        jax.ShapeDtypeStruct((1, 1), jnp.float32),
    ],
)


_NC = 2
_NS = 16
_NW = _NC * _NS
_BPW = _N // _NW
_CH = 128
_NCH = _BPW // _CH


def _gather_body(w_hbm, idx_hbm, out_hbm, idx_v, rows_v, sem):
    wid = lax.axis_index("s") * _NC + lax.axis_index("c")
    base = wid * _BPW
    for j in range(_NCH):
        pltpu.sync_copy(idx_hbm.at[pl.ds(base + j * _CH, _CH)], idx_v.at[j])
    copies = [
        pltpu.async_copy(w_hbm.at[idx_v.at[j]],
                         rows_v.at[pl.ds(j * _CH, _CH)], sem)
        for j in range(_NCH)
    ]
    for c in copies:
        c.wait()
    pltpu.sync_copy(rows_v, out_hbm.at[pl.ds(base, _BPW)])


@functools.cache
def _make_gather_call():
    return pl.kernel(
        _gather_body,
        mesh=plsc.VectorSubcoreMesh(core_axis_name="c", subcore_axis_name="s"),
        compiler_params=pltpu.CompilerParams(use_tc_tiling_on_sc=False),
        out_type=jax.ShapeDtypeStruct((_N, _D), jnp.float32),
        scratch_types=[
            pltpu.VMEM((_NCH, _CH), jnp.int32),
            pltpu.VMEM((_BPW, _D), jnp.float32),
            pltpu.SemaphoreType.DMA,
        ],
    )


def kernel(x, W):
    xf = x.reshape(_N, _D)
    idx3, loss_sum = _argmin_call(xf, W)
    idx = idx3.reshape(_N)
    q = _make_gather_call()(W, idx)
    quantized = x + (q.reshape(x.shape) - x)
    mean_sq = loss_sum[0, 0] / (_N * _D)
    loss = mean_sq + _COMMIT * mean_sq
    return quantized, loss, idx.reshape(_N, 1)


# speedup vs baseline: 1.2321x; 1.0482x over previous
import functools

import jax
import jax.numpy as jnp
from jax import lax
from jax.experimental import pallas as pl
from jax.experimental.pallas import tpu as pltpu
from jax.experimental.pallas import tpu_sc as plsc

_K = 8192
_D = 32
_N = 8192
_TB = 1024
_NB = _N // _TB
_COMMIT = 0.25


_NBLK = 4
_BLK = _K // _NBLK


def _argmin_body(x_ref, w_ref, idx_ref, loss_ref):
    i = pl.program_id(0)
    xb = x_ref[...]
    w = w_ref[...]
    xb16 = xb.astype(jnp.bfloat16)
    s = jnp.sum(xb * xb, axis=1)
    wsq = jnp.sum(w * w, axis=1)
    mm = lax.dot_general(xb16, w, (((1,), (1,)), ((), ())),
                         preferred_element_type=jnp.float32)
    dist = (s[:, None] + wsq[None, :]) - 2.0 * mm
    accv = acci = vsel = None
    for b in range(_NBLK):
        db = dist[:, b * _BLK:(b + 1) * _BLK]
        mv = jnp.min(db, axis=1)
        ids = lax.broadcasted_iota(jnp.int32, (_TB, _BLK), 1) + (b * _BLK)
        im = jnp.min(jnp.where(db == mv[:, None], ids, jnp.int32(2**31 - 1)),
                     axis=1)
        mvb = mv.astype(jnp.bfloat16).astype(jnp.float32)
        if b == 0:
            accv, acci, vsel = mvb, im, mv
        else:
            repl = mv < accv
            accv = jnp.where(repl, mvb, accv)
            acci = jnp.where(repl, im, acci)
            vsel = jnp.where(repl, mv, vsel)
    idx_ref[0, 0, :] = acci

    @pl.when(i == 0)
    def _():
        loss_ref[...] = jnp.zeros((1, 1), jnp.float32)

    loss_ref[...] += jnp.sum(vsel.reshape(1, _TB), axis=1, keepdims=True)


_argmin_call = pl.pallas_call(
    _argmin_body,
    grid=(_NB,),
    in_specs=[
        pl.BlockSpec((_TB, _D), lambda i: (i, 0)),
        pl.BlockSpec((_K, _D), lambda i: (0, 0)),
    ],
    out_specs=[
        pl.BlockSpec((1, 1, _TB), lambda i: (i, 0, 0)),
        pl.BlockSpec((1, 1), lambda i: (0, 0)),
    ],
    out_shape=[
        jax.ShapeDtypeStruct((_NB, 1, _TB), jnp.int32),
        jax.ShapeDtypeStruct((1, 1), jnp.float32),
    ],
)


_NC = 2
_NS = 16
_NW = _NC * _NS
_BPW = _N // _NW
_CH = 128
_NCH = _BPW // _CH


def _gather_body(w_hbm, idx_hbm, out_hbm, idx_v, rows_v, sem):
    wid = lax.axis_index("s") * _NC + lax.axis_index("c")
    base = wid * _BPW
    for j in range(_NCH):
        pltpu.sync_copy(idx_hbm.at[pl.ds(base + j * _CH, _CH)], idx_v.at[j])
    copies = [
        pltpu.async_copy(w_hbm.at[idx_v.at[j]],
                         rows_v.at[pl.ds(j * _CH, _CH)], sem)
        for j in range(_NCH)
    ]
    for c in copies:
        c.wait()
    pltpu.sync_copy(rows_v, out_hbm.at[pl.ds(base, _BPW)])


@functools.cache
def _make_gather_call():
    return pl.kernel(
        _gather_body,
        mesh=plsc.VectorSubcoreMesh(core_axis_name="c", subcore_axis_name="s"),
        compiler_params=pltpu.CompilerParams(use_tc_tiling_on_sc=False),
        out_type=jax.ShapeDtypeStruct((_N, _D), jnp.float32),
        scratch_types=[
            pltpu.VMEM((_NCH, _CH), jnp.int32),
            pltpu.VMEM((_BPW, _D), jnp.float32),
            pltpu.SemaphoreType.DMA,
        ],
    )


def kernel(x, W):
    xf = x.reshape(_N, _D)
    idx3, loss_sum = _argmin_call(xf, W)
    idx = idx3.reshape(_N)
    q = _make_gather_call()(W, idx)
    quantized = x + (q.reshape(x.shape) - x)
    mean_sq = loss_sum[0, 0] / (_N * _D)
    loss = mean_sq + _COMMIT * mean_sq
    return quantized, loss, idx.reshape(_N, 1)
